# baseline (speedup 1.0000x reference)
import functools

import jax
import jax.numpy as jnp
from jax import lax
from jax.experimental import pallas as pl
from jax.experimental.pallas import tpu as pltpu
from jax.experimental.pallas import tpu_sc as plsc

_VOCAB = 100000
_EMBED_DIM = 162
_BATCH = 4096
_SEQ = 50
_B = _BATCH * _SEQ

_NC = 2
_NS = 16
_NW = _NC * _NS
_CHUNK = 128
_PER_W = _B // _NW
_NCHUNK = _PER_W // _CHUNK


def _sc_gather(nid_flat, table):
    mesh = plsc.VectorSubcoreMesh(core_axis_name="c", subcore_axis_name="s")

    @functools.partial(
        pl.kernel,
        out_type=jax.ShapeDtypeStruct((_B, _EMBED_DIM), jnp.float32),
        mesh=mesh,
        scratch_types=[
            pltpu.VMEM((_NCHUNK, _CHUNK), jnp.int32),
            pltpu.VMEM((_CHUNK, _EMBED_DIM), jnp.float32),
            pltpu.SemaphoreType.DMA,
        ],
        compiler_params=pltpu.CompilerParams(use_tc_tiling_on_sc=False),
    )
    def k(idx_hbm, table_hbm, out_hbm, idx_v, rows_v, sem):
        wid = lax.axis_index("s") * _NC + lax.axis_index("c")
        base = wid * _PER_W
        pltpu.sync_copy(idx_hbm.at[wid], idx_v)

        @pl.loop(0, _NCHUNK)
        def _(j):
            pltpu.async_copy(table_hbm.at[idx_v.at[j]], rows_v, sem).wait()
            pltpu.sync_copy(
                rows_v, out_hbm.at[pl.ds(base + j * _CHUNK, _CHUNK)]
            )

    return k(nid_flat, table)


def kernel(nid, table):
    nid_flat = nid.reshape(_NW, _NCHUNK, _CHUNK)
    out = _sc_gather(nid_flat, table)
    return out.reshape(_BATCH, _SEQ, _EMBED_DIM)


# speedup vs baseline: 1.0283x; 1.0283x over previous
import functools

import jax
import jax.numpy as jnp
from jax import lax
from jax.experimental import pallas as pl
from jax.experimental.pallas import tpu as pltpu
from jax.experimental.pallas import tpu_sc as plsc

_VOCAB = 100000
_D = 162
_DP = 168
_BATCH = 4096
_SEQ = 50
_B = _BATCH * _SEQ

_NC = 2
_NS = 16
_NW = _NC * _NS
_CHUNK = 128
_PER_W = _B // _NW
_NCHUNK = _PER_W // _CHUNK
_NBUF = 2


def _sc_gather(nid_w, table_pad):
    mesh = plsc.VectorSubcoreMesh(core_axis_name="c", subcore_axis_name="s")

    @functools.partial(
        pl.kernel,
        out_type=jax.ShapeDtypeStruct((_B, _DP), jnp.float32),
        mesh=mesh,
        scratch_types=[
            pltpu.VMEM((_NCHUNK, _CHUNK), jnp.int32),
            *[pltpu.VMEM((_CHUNK, _DP), jnp.float32) for _ in range(_NBUF)],
            *[pltpu.SemaphoreType.DMA for _ in range(2 * _NBUF)],
        ],
        compiler_params=pltpu.CompilerParams(use_tc_tiling_on_sc=False),
    )
    def k(idx_hbm, table_hbm, out_hbm, idx_v, *rest):
        bufs = rest[:_NBUF]
        gsem = rest[_NBUF : 2 * _NBUF]
        osem = rest[2 * _NBUF : 3 * _NBUF]
        wid = lax.axis_index("s") * _NC + lax.axis_index("c")
        base = wid * _PER_W
        pltpu.sync_copy(idx_hbm.at[wid], idx_v)

        for b in range(_NBUF):
            pltpu.async_copy(table_hbm.at[idx_v.at[b]], bufs[b], gsem[b])

        @pl.loop(0, _NCHUNK // _NBUF)
        def _(j):
            c0 = j * _NBUF
            for b in range(_NBUF):
                c = c0 + b
                pltpu.make_async_copy(
                    table_hbm.at[idx_v.at[0]], bufs[b], gsem[b]
                ).wait()
                out_slice = out_hbm.at[pl.ds(base + c * _CHUNK, _CHUNK)]
                pltpu.async_copy(bufs[b], out_slice, osem[b]).wait()

                @pl.when(c + _NBUF < _NCHUNK)
                def _():
                    pltpu.async_copy(
                        table_hbm.at[idx_v.at[c + _NBUF]], bufs[b], gsem[b]
                    )

    return k(nid_w, table_pad)


def kernel(nid, table):
    nid_w = nid.reshape(_NW, _NCHUNK, _CHUNK)
    table_pad = jnp.pad(table, ((0, 0), (0, _DP - _D)))
    out = _sc_gather(nid_w, table_pad)
    return out[:, :_D].reshape(_BATCH, _SEQ, _D)


# speedup vs baseline: 1.1884x; 1.1557x over previous
import functools

import jax
import jax.numpy as jnp
from jax import lax
from jax.experimental import pallas as pl
from jax.experimental.pallas import tpu as pltpu
from jax.experimental.pallas import tpu_sc as plsc

_VOCAB = 100000
_D = 162
_DP = 168
_BATCH = 4096
_SEQ = 50
_B = _BATCH * _SEQ

_NC = 2
_NS = 16
_NW = _NC * _NS
_CHUNK = 128
_PER_W = _B // _NW
_NCHUNK = _PER_W // _CHUNK
_NBUF = 2


def _sc_gather(nid_w, table_pad):
    mesh = plsc.VectorSubcoreMesh(core_axis_name="c", subcore_axis_name="s")

    @functools.partial(
        pl.kernel,
        out_type=jax.ShapeDtypeStruct((_B, _DP), jnp.float32),
        mesh=mesh,
        scratch_types=[
            pltpu.VMEM((_NCHUNK, _CHUNK), jnp.int32),
            *[pltpu.VMEM((_CHUNK, _DP), jnp.float32) for _ in range(_NBUF)],
            *[pltpu.SemaphoreType.DMA for _ in range(2 * _NBUF)],
        ],
        compiler_params=pltpu.CompilerParams(use_tc_tiling_on_sc=False),
    )
    def k(idx_hbm, table_hbm, out_hbm, idx_v, *rest):
        bufs = rest[:_NBUF]
        gsem = rest[_NBUF : 2 * _NBUF]
        osem = rest[2 * _NBUF : 3 * _NBUF]
        wid = lax.axis_index("s") * _NC + lax.axis_index("c")
        base = wid * _PER_W
        pltpu.sync_copy(idx_hbm.at[wid], idx_v)

        for b in range(_NBUF):
            pltpu.async_copy(table_hbm.at[idx_v.at[b]], bufs[b], gsem[b])

        @pl.loop(0, _NCHUNK // _NBUF)
        def _(j):
            c0 = j * _NBUF
            for b in range(_NBUF):
                c = c0 + b
                pltpu.make_async_copy(
                    table_hbm.at[idx_v.at[0]], bufs[b], gsem[b]
                ).wait()
                out_slice = out_hbm.at[pl.ds(base + c * _CHUNK, _CHUNK)]
                pltpu.async_copy(bufs[b], out_slice, osem[b]).wait()

                @pl.when(c + _NBUF < _NCHUNK)
                def _():
                    pltpu.async_copy(
                        table_hbm.at[idx_v.at[c + _NBUF]], bufs[b], gsem[b]
                    )

    return k(nid_w, table_pad)


def _tc_pad(table):
    rows_blk = 2000
    grid = _VOCAB // rows_blk

    def body(t_ref, o_ref):
        o_ref[:, : _D] = t_ref[...]
        o_ref[:, _D:] = jnp.zeros((rows_blk, _DP - _D), jnp.float32)

    return pl.pallas_call(
        body,
        grid=(grid,),
        in_specs=[pl.BlockSpec((rows_blk, _D), lambda i: (i, 0))],
        out_specs=pl.BlockSpec((rows_blk, _DP), lambda i: (i, 0)),
        out_shape=jax.ShapeDtypeStruct((_VOCAB, _DP), jnp.float32),
    )(table)


def _tc_depad(out_pad):
    rows_blk = 2048
    grid = _B // rows_blk

    def body(p_ref, o_ref):
        o_ref[...] = p_ref[:, : _D]

    return pl.pallas_call(
        body,
        grid=(grid,),
        in_specs=[pl.BlockSpec((rows_blk, _DP), lambda i: (i, 0))],
        out_specs=pl.BlockSpec((rows_blk, _D), lambda i: (i, 0)),
        out_shape=jax.ShapeDtypeStruct((_B, _D), jnp.float32),
    )(out_pad)


def kernel(nid, table):
    nid_w = nid.reshape(_NW, _NCHUNK, _CHUNK)
    table_pad = _tc_pad(table)
    out = _sc_gather(nid_w, table_pad)
    return _tc_depad(out).reshape(_BATCH, _SEQ, _D)


# speedup vs baseline: 2.0625x; 1.7354x over previous
import functools

import jax
import jax.numpy as jnp
from jax import lax
from jax.experimental import pallas as pl
from jax.experimental.pallas import tpu as pltpu
from jax.experimental.pallas import tpu_sc as plsc

_VOCAB = 100000
_D = 162
_DP = 256
_BATCH = 4096
_SEQ = 50
_B = _BATCH * _SEQ

_NC = 2
_NS = 16
_NW = _NC * _NS
_CHUNK = 128
_PER_W = _B // _NW
_NCHUNK = _PER_W // _CHUNK
_NBUF = 2


def _sc_gather(nid_flat, table_pad):
    mesh = plsc.VectorSubcoreMesh(core_axis_name="c", subcore_axis_name="s")

    @functools.partial(
        pl.kernel,
        out_type=jax.ShapeDtypeStruct((_B, _DP), jnp.float32),
        mesh=mesh,
        scratch_types=[
            *[pltpu.VMEM((_CHUNK,), jnp.int32) for _ in range(_NBUF)],
            *[pltpu.VMEM((_CHUNK, _DP), jnp.float32) for _ in range(_NBUF)],
            *[pltpu.SemaphoreType.DMA for _ in range(2 * _NBUF)],
        ],
        compiler_params=pltpu.CompilerParams(use_tc_tiling_on_sc=True),
    )
    def k(idx_hbm, table_hbm, out_hbm, *rest):
        idxb = rest[:_NBUF]
        bufs = rest[_NBUF : 2 * _NBUF]
        gsem = rest[2 * _NBUF : 3 * _NBUF]
        osem = rest[3 * _NBUF : 4 * _NBUF]
        wid = lax.axis_index("s") * _NC + lax.axis_index("c")
        base = wid * _PER_W

        def gather(c, slot):
            pltpu.sync_copy(
                idx_hbm.at[pl.ds(base + c * _CHUNK, _CHUNK)], idxb[slot]
            )
            pltpu.async_copy(table_hbm.at[idxb[slot]], bufs[slot], gsem[slot])

        def gwait(slot):
            pltpu.make_async_copy(
                table_hbm.at[idxb[slot]], bufs[slot], gsem[slot]
            ).wait()

        def copyout(c, slot):
            pltpu.async_copy(
                bufs[slot],
                out_hbm.at[pl.ds(base + c * _CHUNK, _CHUNK)],
                osem[slot],
            ).wait()

        for b in range(_NBUF):
            gather(b, b)

        @pl.loop(0, _NCHUNK)
        def _(c):
            for b in range(_NBUF):
                @pl.when(c % _NBUF == b)
                def _():
                    gwait(b)
                    copyout(c, b)

                    @pl.when(c + _NBUF < _NCHUNK)
                    def _():
                        gather(c + _NBUF, b)

    return k(nid_flat, table_pad)


def _tc_pad(table):
    rows_blk = 2000
    grid = _VOCAB // rows_blk

    def body(t_ref, o_ref):
        o_ref[:, : _D] = t_ref[...]
        o_ref[:, _D:] = jnp.zeros((rows_blk, _DP - _D), jnp.float32)

    return pl.pallas_call(
        body,
        grid=(grid,),
        in_specs=[pl.BlockSpec((rows_blk, _D), lambda i: (i, 0))],
        out_specs=pl.BlockSpec((rows_blk, _DP), lambda i: (i, 0)),
        out_shape=jax.ShapeDtypeStruct((_VOCAB, _DP), jnp.float32),
    )(table)


def _tc_depad(out_pad):
    b_blk = 32
    rows_blk = b_blk * _SEQ
    grid = _B // rows_blk

    def body(p_ref, o_ref):
        o_ref[...] = p_ref[...].reshape(b_blk, _SEQ, _DP)[:, :, : _D]

    return pl.pallas_call(
        body,
        grid=(grid,),
        in_specs=[pl.BlockSpec((rows_blk, _DP), lambda i: (i, 0))],
        out_specs=pl.BlockSpec((b_blk, _SEQ, _D), lambda i: (i, 0, 0)),
        out_shape=jax.ShapeDtypeStruct((_BATCH, _SEQ, _D), jnp.float32),
    )(out_pad)


def kernel(nid, table):
    out_pad = _sc_gather(nid.reshape(_B), _tc_pad(table))
    return _tc_depad(out_pad)


# speedup vs baseline: 2.1089x; 1.0225x over previous
import functools

import jax
import jax.numpy as jnp
from jax import lax
from jax.experimental import pallas as pl
from jax.experimental.pallas import tpu as pltpu
from jax.experimental.pallas import tpu_sc as plsc

_VOCAB = 100000
_D = 162
_DP = 256
_BATCH = 4096
_SEQ = 50
_B = _BATCH * _SEQ

_NC = 2
_NS = 16
_NW = _NC * _NS
_CHUNK = 128
_PER_W = _B // _NW
_NCHUNK = _PER_W // _CHUNK
_NBUF = 2


def _sc_gather(nid_flat, table_pad):
    mesh = plsc.VectorSubcoreMesh(core_axis_name="c", subcore_axis_name="s")

    @functools.partial(
        pl.kernel,
        out_type=jax.ShapeDtypeStruct((_B, _DP), jnp.float32),
        mesh=mesh,
        scratch_types=[
            *[pltpu.VMEM((_CHUNK,), jnp.int32) for _ in range(_NBUF)],
            *[pltpu.VMEM((_CHUNK, _DP), jnp.float32) for _ in range(_NBUF)],
            *[pltpu.SemaphoreType.DMA for _ in range(2 * _NBUF)],
        ],
        compiler_params=pltpu.CompilerParams(use_tc_tiling_on_sc=True),
    )
    def k(idx_hbm, table_hbm, out_hbm, *rest):
        idxb = rest[:_NBUF]
        bufs = rest[_NBUF : 2 * _NBUF]
        gsem = rest[2 * _NBUF : 3 * _NBUF]
        osem = rest[3 * _NBUF : 4 * _NBUF]
        wid = lax.axis_index("s") * _NC + lax.axis_index("c")
        base = wid * _PER_W

        def gather(c, slot):
            pltpu.sync_copy(
                idx_hbm.at[pl.ds(base + c * _CHUNK, _CHUNK)], idxb[slot]
            )
            pltpu.async_copy(table_hbm.at[idxb[slot]], bufs[slot], gsem[slot])

        def gwait(slot):
            pltpu.make_async_copy(
                table_hbm.at[idxb[slot]], bufs[slot], gsem[slot]
            ).wait()

        def copyout(c, slot):
            pltpu.async_copy(
                bufs[slot],
                out_hbm.at[pl.ds(base + c * _CHUNK, _CHUNK)],
                osem[slot],
            ).wait()

        for b in range(_NBUF):
            gather(b, b)

        @pl.loop(0, _NCHUNK)
        def _(c):
            for b in range(_NBUF):
                @pl.when(c % _NBUF == b)
                def _():
                    gwait(b)
                    copyout(c, b)

                    @pl.when(c + _NBUF < _NCHUNK)
                    def _():
                        gather(c + _NBUF, b)

    return k(nid_flat, table_pad)


def _tc_pad(table):
    rows_blk = 2000
    grid = _VOCAB // rows_blk

    def body(t_ref, o_ref):
        o_ref[:, : _D] = t_ref[...]
        o_ref[:, _D:] = jnp.zeros((rows_blk, _DP - _D), jnp.float32)

    return pl.pallas_call(
        body,
        grid=(grid,),
        in_specs=[pl.BlockSpec((rows_blk, _D), lambda i: (i, 0))],
        out_specs=pl.BlockSpec((rows_blk, _DP), lambda i: (i, 0)),
        out_shape=jax.ShapeDtypeStruct((_VOCAB, _DP), jnp.float32),
    )(table)


def _tc_depad(out_pad):
    b_blk = 32
    rows_blk = b_blk * _SEQ
    grid = _B // rows_blk

    def body(p_ref, o_ref):
        o_ref[...] = p_ref[...].reshape(b_blk, _SEQ, _DP)[:, :, : _D]

    return pl.pallas_call(
        body,
        grid=(grid,),
        in_specs=[pl.BlockSpec((rows_blk, _DP), lambda i: (i, 0))],
        out_specs=pl.BlockSpec((b_blk, _SEQ, _D), lambda i: (i, 0, 0)),
        out_shape=jax.ShapeDtypeStruct((_BATCH, _SEQ, _D), jnp.float32),
    )(out_pad)


def kernel(nid, table):
    out_pad = _sc_gather(nid.reshape(_B), _tc_pad(table))
    return out_pad[:, :_D].reshape(_BATCH, _SEQ, _D)
